# Initial kernel scaffold; baseline (speedup 1.0000x reference)
import jax
import jax.numpy as jnp
from jax.experimental import pallas as pl


def kernel(previous_inclusion_score, nodes, adjacency_matrix, W_phi, W_theta):
    raise NotImplementedError("write your pallas kernel here")


# speedup vs baseline: 2.3001x; 2.3001x over previous
import jax
import jax.numpy as jnp
from jax.experimental import pallas as pl

N = 4096
TM = 256


def _body(prev_ref, nblk_ref, ntT_ref, a_ref, wphi_ref, wth_ref, out_ref):
    w0 = wth_ref[0, 0]
    w1 = wth_ref[1, 0]
    w2 = wth_ref[2, 0]
    c0 = w0 * w0
    c1 = w1 * w1
    c2 = w2 * w2

    x0 = ntT_ref[0:1, :]
    x1 = ntT_ref[1:2, :]
    x2 = ntT_ref[2:3, :]
    g0 = x0 * c0
    g1 = x1 * c1
    g2 = x2 * c2
    sqj = x0 * g0 + x1 * g1 + x2 * g2

    y0 = nblk_ref[:, 0:1]
    y1 = nblk_ref[:, 1:2]
    y2 = nblk_ref[:, 2:3]
    sqi = y0 * y0 * c0 + y1 * y1 * c1 + y2 * y2 * c2

    cross = y0 * g0 + y1 * g1 + y2 * g2
    z = sqj - (cross + cross)

    mask = a_ref[:, :] != 0
    neg = jnp.float32(-jnp.inf)
    m = jnp.max(jnp.where(mask, z, neg), axis=1, keepdims=True)
    d2 = sqi + m
    maxd = jnp.where(m == neg, neg, jnp.sqrt(jnp.maximum(d2, 0.0)))

    wmean = jnp.mean(wphi_ref[0, :])
    out_ref[:, :] = 0.5 * (prev_ref[:, :] + maxd * wmean)


@jax.jit
def _run(prev, nodes, adjacency, wphi, wth):
    prev2 = prev.reshape(N, 1)
    ntT = nodes.T
    wphi2 = wphi.reshape(1, -1)
    grid = (N // TM,)
    out = pl.pallas_call(
        _body,
        grid=grid,
        in_specs=[
            pl.BlockSpec((TM, 1), lambda i: (i, 0)),
            pl.BlockSpec((TM, 3), lambda i: (i, 0)),
            pl.BlockSpec((3, N), lambda i: (0, 0)),
            pl.BlockSpec((TM, N), lambda i: (i, 0)),
            pl.BlockSpec((1, wphi.shape[0]), lambda i: (0, 0)),
            pl.BlockSpec((3, 1), lambda i: (0, 0)),
        ],
        out_specs=pl.BlockSpec((TM, 1), lambda i: (i, 0)),
        out_shape=jax.ShapeDtypeStruct((N, 1), jnp.float32),
    )(prev2, nodes, ntT, adjacency, wphi2, wth)
    return out.reshape(N)


def kernel(previous_inclusion_score, nodes, adjacency_matrix, W_phi, W_theta):
    return _run(previous_inclusion_score, nodes, adjacency_matrix, W_phi, W_theta)


# speedup vs baseline: 2.5467x; 1.1072x over previous
import jax
import jax.numpy as jnp
from jax.experimental import pallas as pl

N = 4096
TM = 512


def _body(prev_ref, nblk_ref, ntT_ref, a_ref, wphi_ref, wth_ref, out_ref):
    w0 = wth_ref[0, 0]
    w1 = wth_ref[1, 0]
    w2 = wth_ref[2, 0]
    c0 = w0 * w0
    c1 = w1 * w1
    c2 = w2 * w2

    x0 = ntT_ref[0:1, :]
    x1 = ntT_ref[1:2, :]
    x2 = ntT_ref[2:3, :]
    g0 = x0 * (-2.0 * c0)
    g1 = x1 * (-2.0 * c1)
    g2 = x2 * (-2.0 * c2)
    sqj = (x0 * x0 * c0 + x1 * x1 * c1 + x2 * x2 * c2)

    y0 = nblk_ref[:, 0:1]
    y1 = nblk_ref[:, 1:2]
    y2 = nblk_ref[:, 2:3]
    sqi = y0 * y0 * c0 + y1 * y1 * c1 + y2 * y2 * c2

    z = ((sqj + y0 * g0) + y1 * g1) + y2 * g2

    mask = a_ref[:, :] != 0
    neg = jnp.float32(-jnp.inf)
    m = jnp.max(jnp.where(mask, z, neg), axis=1, keepdims=True)
    d2 = sqi + m
    maxd = jnp.where(m == neg, neg, jnp.sqrt(jnp.maximum(d2, 0.0)))

    wmean = jnp.mean(wphi_ref[0, :])
    out_ref[:, :] = 0.5 * (prev_ref[:, :] + maxd * wmean)


@jax.jit
def _run(prev, nodes, adjacency, wphi, wth):
    prev2 = prev.reshape(N, 1)
    ntT = nodes.T
    wphi2 = wphi.reshape(1, -1)
    grid = (N // TM,)
    out = pl.pallas_call(
        _body,
        grid=grid,
        in_specs=[
            pl.BlockSpec((TM, 1), lambda i: (i, 0)),
            pl.BlockSpec((TM, 3), lambda i: (i, 0)),
            pl.BlockSpec((3, N), lambda i: (0, 0)),
            pl.BlockSpec((TM, N), lambda i: (i, 0)),
            pl.BlockSpec((1, wphi.shape[0]), lambda i: (0, 0)),
            pl.BlockSpec((3, 1), lambda i: (0, 0)),
        ],
        out_specs=pl.BlockSpec((TM, 1), lambda i: (i, 0)),
        out_shape=jax.ShapeDtypeStruct((N, 1), jnp.float32),
    )(prev2, nodes, ntT, adjacency, wphi2, wth)
    return out.reshape(N)


def kernel(previous_inclusion_score, nodes, adjacency_matrix, W_phi, W_theta):
    return _run(previous_inclusion_score, nodes, adjacency_matrix, W_phi, W_theta)
